# baseline (speedup 1.0000x reference)
import functools

import jax
import jax.numpy as jnp
from jax import lax
from jax.experimental import pallas as pl
from jax.experimental.pallas import tpu as pltpu
from jax.experimental.pallas import tpu_sc as plsc

N_NODES = 10000
N_EDGES = 320000
DIM = 128
BN_EPS = 1e-5
LEAKY_SLOPE = 0.01

NC, NS = 2, 16
NW = NC * NS
K = 128
NCHUNK = 80
EPW = NCHUNK * K
EPAD = NW * EPW
NPAD = 10240
ROWS_PW = NPAD // NS
NCHUNK2 = 159
EPW2 = NCHUNK2 * K
EPAD2 = NS * EPW2
HDIM = DIM // NC


def _sc_hist_body(dst_hbm, out_hbm, idx_v, ones_v, zbuf_v, hist_sh):
    c = lax.axis_index("c")
    s = lax.axis_index("s")
    w = c * NS + s
    for i in range(K // 16):
        ones_v[pl.ds(16 * i, 16)] = jnp.ones((16,), jnp.float32)
    for i in range(ROWS_PW // 16):
        zbuf_v[pl.ds(16 * i, 16)] = jnp.zeros((16,), jnp.float32)
    pltpu.sync_copy(zbuf_v, hist_sh.at[pl.ds(s * ROWS_PW, ROWS_PW)])
    plsc.subcore_barrier()
    pltpu.sync_copy(dst_hbm.at[w], idx_v)

    def chunk(j, carry):
        pltpu.sync_copy(ones_v, hist_sh.at[idx_v.at[j]], add=True)
        return carry

    lax.fori_loop(0, NCHUNK, chunk, 0)
    plsc.subcore_barrier()
    pltpu.sync_copy(hist_sh.at[pl.ds(s * ROWS_PW, ROWS_PW)], zbuf_v)
    pltpu.sync_copy(zbuf_v, out_hbm.at[c, pl.ds(s * ROWS_PW, ROWS_PW)])


def _sc_agg_body(src_hbm, dst_hbm, y_hbm, out_hbm, sidx_v, didx_v, gidx_v,
                 bufa_v, bufb_v, bufc_v, acc_sh, sema, semb, semc):
    c = lax.axis_index("c")
    s = lax.axis_index("s")
    for r in range(16):
        for i in range(HDIM // 16):
            bufa_v[r, pl.ds(16 * i, 16)] = jnp.zeros((16,), jnp.float32)

    def zinit(t, carry):
        pltpu.sync_copy(bufa_v.at[pl.ds(0, 16)],
                        acc_sh.at[pl.ds(s * ROWS_PW + 16 * t, 16)])
        return carry

    lax.fori_loop(0, ROWS_PW // 16, zinit, 0)
    pltpu.sync_copy(src_hbm.at[s], sidx_v)
    pltpu.sync_copy(dst_hbm.at[s], didx_v)

    def gcomp(j, carry):
        for i in range(K // 16):
            v = sidx_v[j, pl.ds(16 * i, 16)]
            gidx_v[j, pl.ds(16 * i, 16)] = v * 2 + c
        return carry

    lax.fori_loop(0, NCHUNK2, gcomp, 0)
    plsc.subcore_barrier()

    def gather_start(j, buf, sem):
        pltpu.async_copy(y_hbm.at[gidx_v.at[j]], buf, sem)

    def gather_wait(buf, sem):
        pltpu.make_async_copy(y_hbm.at[gidx_v.at[0]], buf, sem).wait()

    gather_start(0, bufa_v, sema)
    gather_start(1, bufb_v, semb)
    gather_start(2, bufc_v, semc)

    def trip(t, carry):
        j = 3 * t
        gather_wait(bufa_v, sema)
        pltpu.sync_copy(bufa_v, acc_sh.at[didx_v.at[j]], add=True)
        gather_start(j + 3, bufa_v, sema)
        gather_wait(bufb_v, semb)
        pltpu.sync_copy(bufb_v, acc_sh.at[didx_v.at[j + 1]], add=True)
        gather_start(j + 4, bufb_v, semb)
        gather_wait(bufc_v, semc)
        pltpu.sync_copy(bufc_v, acc_sh.at[didx_v.at[j + 2]], add=True)
        gather_start(j + 5, bufc_v, semc)
        return carry

    lax.fori_loop(0, NCHUNK2 // 3 - 1, trip, 0)
    gather_wait(bufa_v, sema)
    pltpu.sync_copy(bufa_v, acc_sh.at[didx_v.at[NCHUNK2 - 3]], add=True)
    gather_wait(bufb_v, semb)
    pltpu.sync_copy(bufb_v, acc_sh.at[didx_v.at[NCHUNK2 - 2]], add=True)
    gather_wait(bufc_v, semc)
    pltpu.sync_copy(bufc_v, acc_sh.at[didx_v.at[NCHUNK2 - 1]], add=True)
    plsc.subcore_barrier()

    def writeback(t, carry):
        base = s * ROWS_PW + K * t
        pltpu.sync_copy(acc_sh.at[pl.ds(base, K)], bufa_v)
        pltpu.sync_copy(bufa_v, out_hbm.at[c, pl.ds(base, K)])
        return carry

    lax.fori_loop(0, ROWS_PW // K, writeback, 0)


def _tc_dense_body(fea_ref, w1t_ref, b1_ref, g_ref, be_ref, mu_ref, var_ref,
                   wgt_ref, h_ref, y_ref):
    x = jnp.dot(fea_ref[...], w1t_ref[...], preferred_element_type=jnp.float32)
    x = x + b1_ref[...]
    scale = g_ref[...] * lax.rsqrt(var_ref[...] + BN_EPS)
    x = (x - mu_ref[...]) * scale + be_ref[...]
    x = jnp.where(x >= 0, x, LEAKY_SLOPE * x)
    xw = jnp.dot(x, wgt_ref[...], preferred_element_type=jnp.float32)
    deg = h_ref[0] + h_ref[1] + 1.0
    dis = lax.rsqrt(deg)
    y_ref[...] = dis * xw


def _tc_final_body(p_ref, y_ref, h_ref, bg_ref, o_ref):
    deg = h_ref[0] + h_ref[1] + 1.0
    dis = lax.rsqrt(deg)
    acc = jnp.concatenate([p_ref[0], p_ref[1]], axis=1)
    o_ref[...] = dis * (acc + y_ref[...]) + bg_ref[...]


_mesh = plsc.VectorSubcoreMesh(core_axis_name="c", subcore_axis_name="s",
                               num_cores=NC, num_subcores=NS)

_sc_hist = pl.kernel(
    _sc_hist_body,
    out_type=jax.ShapeDtypeStruct((NC, NPAD), jnp.float32),
    mesh=_mesh,
    scratch_types=[
        pltpu.VMEM((NCHUNK, K), jnp.int32),
        pltpu.VMEM((K,), jnp.float32),
        pltpu.VMEM((ROWS_PW,), jnp.float32),
        pltpu.VMEM_SHARED((NPAD,), jnp.float32),
    ],
)

_sc_agg = pl.kernel(
    _sc_agg_body,
    out_type=jax.ShapeDtypeStruct((NC, NPAD, HDIM), jnp.float32),
    mesh=_mesh,
    scratch_types=[
        pltpu.VMEM((NCHUNK2, K), jnp.int32),
        pltpu.VMEM((NCHUNK2, K), jnp.int32),
        pltpu.VMEM((NCHUNK2, K), jnp.int32),
        pltpu.VMEM((K, HDIM), jnp.float32),
        pltpu.VMEM((K, HDIM), jnp.float32),
        pltpu.VMEM((K, HDIM), jnp.float32),
        pltpu.VMEM_SHARED((NPAD, HDIM), jnp.float32),
        pltpu.SemaphoreType.DMA,
        pltpu.SemaphoreType.DMA,
        pltpu.SemaphoreType.DMA,
    ],
    compiler_params=pltpu.CompilerParams(use_tc_tiling_on_sc=False),
)

_ROWBLK = 1000
_GRID = N_NODES // _ROWBLK

_tc_dense = pl.pallas_call(
    _tc_dense_body,
    grid=(_GRID,),
    in_specs=[
        pl.BlockSpec((_ROWBLK, DIM), lambda i: (i, 0)),
        pl.BlockSpec((DIM, DIM), lambda i: (0, 0)),
        pl.BlockSpec((1, DIM), lambda i: (0, 0)),
        pl.BlockSpec((1, DIM), lambda i: (0, 0)),
        pl.BlockSpec((1, DIM), lambda i: (0, 0)),
        pl.BlockSpec((1, DIM), lambda i: (0, 0)),
        pl.BlockSpec((1, DIM), lambda i: (0, 0)),
        pl.BlockSpec((DIM, DIM), lambda i: (0, 0)),
        pl.BlockSpec((NC, _ROWBLK, 1), lambda i: (0, i, 0)),
    ],
    out_specs=pl.BlockSpec((_ROWBLK, DIM), lambda i: (i, 0)),
    out_shape=jax.ShapeDtypeStruct((N_NODES, DIM), jnp.float32),
)

_tc_final = pl.pallas_call(
    _tc_final_body,
    grid=(_GRID,),
    in_specs=[
        pl.BlockSpec((NC, _ROWBLK, HDIM), lambda i: (0, i, 0)),
        pl.BlockSpec((_ROWBLK, DIM), lambda i: (i, 0)),
        pl.BlockSpec((NC, _ROWBLK, 1), lambda i: (0, i, 0)),
        pl.BlockSpec((1, DIM), lambda i: (0, 0)),
    ],
    out_specs=pl.BlockSpec((_ROWBLK, DIM), lambda i: (i, 0)),
    out_shape=jax.ShapeDtypeStruct((N_NODES, DIM), jnp.float32),
)


@jax.jit
def _impl(fea, edges, W1, b1, bn_gamma, bn_beta, bn_mean, bn_var, Wg, bg):
    spad = jnp.zeros((EPAD2 - N_EDGES,), dtype=edges.dtype)
    dpad = jnp.full((EPAD2 - N_EDGES,), N_NODES, dtype=edges.dtype)
    src = jnp.concatenate([edges[0], spad])
    dst = jnp.concatenate([edges[1], dpad])
    dpad_h = jnp.full((EPAD - N_EDGES,), N_NODES, dtype=edges.dtype)
    dst_h = jnp.concatenate([edges[1], dpad_h])

    hist = _sc_hist(dst_h.reshape(NW, NCHUNK, K))
    hist3 = hist[:, :N_NODES].reshape(NC, N_NODES, 1)
    y = _tc_dense(fea, W1.T, b1.reshape(1, DIM), bn_gamma.reshape(1, DIM),
                  bn_beta.reshape(1, DIM), bn_mean.reshape(1, DIM),
                  bn_var.reshape(1, DIM), Wg.T, hist3)
    parts = _sc_agg(src.reshape(NS, NCHUNK2, K), dst.reshape(NS, NCHUNK2, K),
                    y.reshape(NC * N_NODES, HDIM))
    return _tc_final(parts, y, hist3, bg.reshape(1, DIM))


def kernel(fea, edges, W1, b1, bn_gamma, bn_beta, bn_mean, bn_var, Wg, bg):
    return _impl(fea, edges, W1, b1, bn_gamma, bn_beta, bn_mean, bn_var, Wg, bg)


# speedup vs baseline: 1.1727x; 1.1727x over previous
import functools

import jax
import jax.numpy as jnp
from jax import lax
from jax.experimental import pallas as pl
from jax.experimental.pallas import tpu as pltpu
from jax.experimental.pallas import tpu_sc as plsc

N_NODES = 10000
N_EDGES = 320000
DIM = 128
BN_EPS = 1e-5
LEAKY_SLOPE = 0.01

NC, NS = 2, 16
NW = NC * NS
K = 128
NCHUNK = 80
EPW = NCHUNK * K
EPAD = NW * EPW
NPAD = 10240
ROWS_PW = NPAD // NS
NCHUNK2 = 158
NPAIR = NCHUNK2 // 2
EPW2 = NCHUNK2 * K
EPAD2 = NS * EPW2
HDIM = DIM // NC


def _sc_hist_body(dst_hbm, out_hbm, idx_v, ones_v, zbuf_v, hist_sh):
    c = lax.axis_index("c")
    s = lax.axis_index("s")
    w = c * NS + s
    for i in range(K // 16):
        ones_v[pl.ds(16 * i, 16)] = jnp.ones((16,), jnp.float32)
    for i in range(ROWS_PW // 16):
        zbuf_v[pl.ds(16 * i, 16)] = jnp.zeros((16,), jnp.float32)
    pltpu.sync_copy(zbuf_v, hist_sh.at[pl.ds(s * ROWS_PW, ROWS_PW)])
    plsc.subcore_barrier()
    pltpu.sync_copy(dst_hbm.at[w], idx_v)

    def chunk(j, carry):
        pltpu.sync_copy(ones_v, hist_sh.at[idx_v.at[j]], add=True)
        return carry

    lax.fori_loop(0, NCHUNK, chunk, 0)
    plsc.subcore_barrier()
    pltpu.sync_copy(hist_sh.at[pl.ds(s * ROWS_PW, ROWS_PW)], zbuf_v)
    pltpu.sync_copy(zbuf_v, out_hbm.at[c, pl.ds(s * ROWS_PW, ROWS_PW)])


def _sc_agg_body(src_hbm, dst_hbm, y_hbm, out_hbm, sidx_v, didx_v,
                 bufa_v, bufb_v, acc_sh, sema, semb):
    c = lax.axis_index("c")
    s = lax.axis_index("s")
    for r in range(16):
        for i in range(HDIM // 16):
            bufa_v[r, pl.ds(16 * i, 16)] = jnp.zeros((16,), jnp.float32)

    def zinit(t, carry):
        pltpu.sync_copy(bufa_v.at[pl.ds(0, 16)],
                        acc_sh.at[pl.ds(s * ROWS_PW + 16 * t, 16)])
        return carry

    lax.fori_loop(0, ROWS_PW // 16, zinit, 0)
    pltpu.sync_copy(src_hbm.at[s], sidx_v)
    pltpu.sync_copy(dst_hbm.at[s], didx_v)

    def gcomp(j, carry):
        for i in range(K // 16):
            v = sidx_v[j, pl.ds(16 * i, 16)]
            sidx_v[j, pl.ds(16 * i, 16)] = v * 2 + c
        return carry

    lax.fori_loop(0, NCHUNK2, gcomp, 0)
    plsc.subcore_barrier()

    def gather_start(j, buf, sem):
        pltpu.async_copy(y_hbm.at[sidx_v.at[j]], buf, sem)

    def gather_wait(buf, sem):
        pltpu.make_async_copy(y_hbm.at[sidx_v.at[0]], buf, sem).wait()

    def scat(j, buf):
        pltpu.sync_copy(buf, acc_sh.at[didx_v.at[j]], add=True)

    gather_start(0, bufa_v, sema)
    gather_start(1, bufb_v, semb)

    def pair(t, carry):
        j0 = 2 * t
        gather_wait(bufa_v, sema)
        scat(j0, bufa_v)
        gather_start(j0 + 2, bufa_v, sema)
        gather_wait(bufb_v, semb)
        scat(j0 + 1, bufb_v)
        gather_start(j0 + 3, bufb_v, semb)
        return carry

    lax.fori_loop(0, NCHUNK2 // 2 - 1, pair, 0)
    gather_wait(bufa_v, sema)
    scat(NCHUNK2 - 2, bufa_v)
    gather_wait(bufb_v, semb)
    scat(NCHUNK2 - 1, bufb_v)
    plsc.subcore_barrier()

    def writeback(t, carry):
        base = s * ROWS_PW + K * t
        pltpu.sync_copy(acc_sh.at[pl.ds(base, K)], bufa_v)
        pltpu.sync_copy(bufa_v, out_hbm.at[c, pl.ds(base, K)])
        return carry

    lax.fori_loop(0, ROWS_PW // K, writeback, 0)


def _tc_dense_body(fea_ref, w1t_ref, b1_ref, g_ref, be_ref, mu_ref, var_ref,
                   wgt_ref, h_ref, y_ref):
    x = jnp.dot(fea_ref[...], w1t_ref[...], preferred_element_type=jnp.float32)
    x = x + b1_ref[...]
    scale = g_ref[...] * lax.rsqrt(var_ref[...] + BN_EPS)
    x = (x - mu_ref[...]) * scale + be_ref[...]
    x = jnp.where(x >= 0, x, LEAKY_SLOPE * x)
    xw = jnp.dot(x, wgt_ref[...], preferred_element_type=jnp.float32)
    deg = h_ref[0] + h_ref[1] + 1.0
    dis = lax.rsqrt(deg)
    y_ref[...] = dis * xw


def _tc_final_body(p_ref, y_ref, h_ref, bg_ref, o_ref):
    deg = h_ref[0] + h_ref[1] + 1.0
    dis = lax.rsqrt(deg)
    acc = jnp.concatenate([p_ref[0], p_ref[1]], axis=1)
    o_ref[...] = dis * (acc + y_ref[...]) + bg_ref[...]


_mesh = plsc.VectorSubcoreMesh(core_axis_name="c", subcore_axis_name="s",
                               num_cores=NC, num_subcores=NS)

_sc_hist = pl.kernel(
    _sc_hist_body,
    out_type=jax.ShapeDtypeStruct((NC, NPAD), jnp.float32),
    mesh=_mesh,
    scratch_types=[
        pltpu.VMEM((NCHUNK, K), jnp.int32),
        pltpu.VMEM((K,), jnp.float32),
        pltpu.VMEM((ROWS_PW,), jnp.float32),
        pltpu.VMEM_SHARED((NPAD,), jnp.float32),
    ],
)

_sc_agg = pl.kernel(
    _sc_agg_body,
    out_type=jax.ShapeDtypeStruct((NC, NPAD, HDIM), jnp.float32),
    mesh=_mesh,
    scratch_types=[
        pltpu.VMEM((NCHUNK2, K), jnp.int32),
        pltpu.VMEM((NCHUNK2, K), jnp.int32),
        pltpu.VMEM((K, HDIM), jnp.float32),
        pltpu.VMEM((K, HDIM), jnp.float32),
        pltpu.VMEM_SHARED((NPAD, HDIM), jnp.float32),
        pltpu.SemaphoreType.DMA,
        pltpu.SemaphoreType.DMA,
    ],
    compiler_params=pltpu.CompilerParams(use_tc_tiling_on_sc=False),
)

_ROWBLK = 1000
_GRID = N_NODES // _ROWBLK

_tc_dense = pl.pallas_call(
    _tc_dense_body,
    grid=(_GRID,),
    in_specs=[
        pl.BlockSpec((_ROWBLK, DIM), lambda i: (i, 0)),
        pl.BlockSpec((DIM, DIM), lambda i: (0, 0)),
        pl.BlockSpec((1, DIM), lambda i: (0, 0)),
        pl.BlockSpec((1, DIM), lambda i: (0, 0)),
        pl.BlockSpec((1, DIM), lambda i: (0, 0)),
        pl.BlockSpec((1, DIM), lambda i: (0, 0)),
        pl.BlockSpec((1, DIM), lambda i: (0, 0)),
        pl.BlockSpec((DIM, DIM), lambda i: (0, 0)),
        pl.BlockSpec((NC, _ROWBLK, 1), lambda i: (0, i, 0)),
    ],
    out_specs=pl.BlockSpec((_ROWBLK, DIM), lambda i: (i, 0)),
    out_shape=jax.ShapeDtypeStruct((N_NODES, DIM), jnp.float32),
)

_tc_final = pl.pallas_call(
    _tc_final_body,
    grid=(_GRID,),
    in_specs=[
        pl.BlockSpec((NC, _ROWBLK, HDIM), lambda i: (0, i, 0)),
        pl.BlockSpec((_ROWBLK, DIM), lambda i: (i, 0)),
        pl.BlockSpec((NC, _ROWBLK, 1), lambda i: (0, i, 0)),
        pl.BlockSpec((1, DIM), lambda i: (0, 0)),
    ],
    out_specs=pl.BlockSpec((_ROWBLK, DIM), lambda i: (i, 0)),
    out_shape=jax.ShapeDtypeStruct((N_NODES, DIM), jnp.float32),
)


@jax.jit
def _impl(fea, edges, W1, b1, bn_gamma, bn_beta, bn_mean, bn_var, Wg, bg):
    spad = jnp.zeros((EPAD2 - N_EDGES,), dtype=edges.dtype)
    dpad = jnp.full((EPAD2 - N_EDGES,), N_NODES, dtype=edges.dtype)
    src = jnp.concatenate([edges[0], spad])
    dst = jnp.concatenate([edges[1], dpad])
    dpad_h = jnp.full((EPAD - N_EDGES,), N_NODES, dtype=edges.dtype)
    dst_h = jnp.concatenate([edges[1], dpad_h])

    hist = _sc_hist(dst_h.reshape(NW, NCHUNK, K))
    hist3 = hist[:, :N_NODES].reshape(NC, N_NODES, 1)
    y = _tc_dense(fea, W1.T, b1.reshape(1, DIM), bn_gamma.reshape(1, DIM),
                  bn_beta.reshape(1, DIM), bn_mean.reshape(1, DIM),
                  bn_var.reshape(1, DIM), Wg.T, hist3)
    parts = _sc_agg(src.reshape(NS, NCHUNK2, K), dst.reshape(NS, NCHUNK2, K),
                    y.reshape(NC * N_NODES, HDIM))
    return _tc_final(parts, y, hist3, bg.reshape(1, DIM))


def kernel(fea, edges, W1, b1, bn_gamma, bn_beta, bn_mean, bn_var, Wg, bg):
    return _impl(fea, edges, W1, b1, bn_gamma, bn_beta, bn_mean, bn_var, Wg, bg)


# speedup vs baseline: 1.5590x; 1.3294x over previous
import functools

import jax
import jax.numpy as jnp
from jax import lax
from jax.experimental import pallas as pl
from jax.experimental.pallas import tpu as pltpu
from jax.experimental.pallas import tpu_sc as plsc

N_NODES = 10000
N_EDGES = 320000
DIM = 128
BN_EPS = 1e-5
LEAKY_SLOPE = 0.01

NC, NS = 2, 16
NW = NC * NS
K = 128
NCHUNK = 80
EPW = NCHUNK * K
EPAD = NW * EPW
NPAD = 10240
ROWS_PW = NPAD // NS
NCHUNK2 = 158
NPHASE = 2
CPQ = NCHUNK2 // NPHASE
EPW2 = NCHUNK2 * K
EPAD2 = NS * EPW2
HDIM = DIM // NC


def _sc_hist_body(dst_hbm, out_hbm, idx_v, ones_v, zbuf_v, hist_sh):
    c = lax.axis_index("c")
    s = lax.axis_index("s")
    w = c * NS + s
    for i in range(K // 16):
        ones_v[pl.ds(16 * i, 16)] = jnp.ones((16,), jnp.float32)
    for i in range(ROWS_PW // 16):
        zbuf_v[pl.ds(16 * i, 16)] = jnp.zeros((16,), jnp.float32)
    pltpu.sync_copy(zbuf_v, hist_sh.at[pl.ds(s * ROWS_PW, ROWS_PW)])
    plsc.subcore_barrier()
    pltpu.sync_copy(dst_hbm.at[w], idx_v)

    def chunk(j, carry):
        pltpu.sync_copy(ones_v, hist_sh.at[idx_v.at[j]], add=True)
        return carry

    lax.fori_loop(0, NCHUNK, chunk, 0)
    plsc.subcore_barrier()
    pltpu.sync_copy(hist_sh.at[pl.ds(s * ROWS_PW, ROWS_PW)], zbuf_v)
    pltpu.sync_copy(zbuf_v, out_hbm.at[c, pl.ds(s * ROWS_PW, ROWS_PW)])


def _sc_agg_body(src_hbm, dst_hbm, y_hbm, out_hbm, sidx_v, didx_v,
                 bufa_v, bufb_v, y_sh, acc_sh, sema, semb):
    c = lax.axis_index("c")
    s = lax.axis_index("s")
    for r in range(16):
        for i in range(HDIM // 16):
            bufa_v[r, pl.ds(16 * i, 16)] = jnp.zeros((16,), jnp.float32)

    def zinit(t, carry):
        pltpu.sync_copy(bufa_v.at[pl.ds(0, 16)],
                        acc_sh.at[pl.ds(s * ROWS_PW + 16 * t, 16)])
        return carry

    lax.fori_loop(0, ROWS_PW // 16, zinit, 0)

    def ystage(t, carry):
        base = s * (N_NODES // NS) + 125 * t
        pltpu.sync_copy(y_hbm.at[c, pl.ds(base, 125)],
                        bufb_v.at[pl.ds(0, 125)])
        pltpu.sync_copy(bufb_v.at[pl.ds(0, 125)], y_sh.at[pl.ds(base, 125)])
        return carry

    lax.fori_loop(0, (N_NODES // NS) // 125, ystage, 0)
    plsc.subcore_barrier()

    def gather_start(j, buf, sem):
        pltpu.async_copy(y_sh.at[sidx_v.at[j]], buf, sem)

    def gather_wait(buf, sem):
        pltpu.make_async_copy(y_sh.at[sidx_v.at[0]], buf, sem).wait()

    def scat(j, buf):
        pltpu.sync_copy(buf, acc_sh.at[didx_v.at[j]], add=True)

    for p in range(NPHASE):
        pltpu.sync_copy(src_hbm.at[s, p], sidx_v)
        pltpu.sync_copy(dst_hbm.at[s, p], didx_v)
        gather_start(0, bufa_v, sema)
        gather_start(1, bufb_v, semb)

        def pair(t, carry):
            j0 = 2 * t
            gather_wait(bufa_v, sema)
            scat(j0, bufa_v)
            gather_start(j0 + 2, bufa_v, sema)
            gather_wait(bufb_v, semb)
            scat(j0 + 1, bufb_v)
            gather_start(j0 + 3, bufb_v, semb)
            return carry

        lax.fori_loop(0, CPQ // 2 - 1, pair, 0)
        gather_wait(bufa_v, sema)
        scat(CPQ - 3, bufa_v)
        gather_start(CPQ - 1, bufa_v, sema)
        gather_wait(bufb_v, semb)
        scat(CPQ - 2, bufb_v)
        gather_wait(bufa_v, sema)
        scat(CPQ - 1, bufa_v)
    plsc.subcore_barrier()

    def writeback(t, carry):
        base = s * ROWS_PW + K * t
        pltpu.sync_copy(acc_sh.at[pl.ds(base, K)], bufa_v)
        pltpu.sync_copy(bufa_v, out_hbm.at[c, pl.ds(base, K)])
        return carry

    lax.fori_loop(0, ROWS_PW // K, writeback, 0)


def _tc_dense_body(fea_ref, w1t_ref, b1_ref, g_ref, be_ref, mu_ref, var_ref,
                   wgt_ref, h_ref, y_ref):
    x = jnp.dot(fea_ref[...], w1t_ref[...], preferred_element_type=jnp.float32)
    x = x + b1_ref[...]
    scale = g_ref[...] * lax.rsqrt(var_ref[...] + BN_EPS)
    x = (x - mu_ref[...]) * scale + be_ref[...]
    x = jnp.where(x >= 0, x, LEAKY_SLOPE * x)
    xw = jnp.dot(x, wgt_ref[...], preferred_element_type=jnp.float32)
    deg = h_ref[0] + h_ref[1] + 1.0
    dis = lax.rsqrt(deg)
    y = dis * xw
    y_ref[0] = y[:, :HDIM]
    y_ref[1] = y[:, HDIM:]


def _tc_final_body(p_ref, y_ref, h_ref, bg_ref, o_ref):
    deg = h_ref[0] + h_ref[1] + 1.0
    dis = lax.rsqrt(deg)
    acc = jnp.concatenate([p_ref[0] + y_ref[0], p_ref[1] + y_ref[1]], axis=1)
    o_ref[...] = dis * acc + bg_ref[...]


_mesh = plsc.VectorSubcoreMesh(core_axis_name="c", subcore_axis_name="s",
                               num_cores=NC, num_subcores=NS)

_sc_hist = pl.kernel(
    _sc_hist_body,
    out_type=jax.ShapeDtypeStruct((NC, NPAD), jnp.float32),
    mesh=_mesh,
    scratch_types=[
        pltpu.VMEM((NCHUNK, K), jnp.int32),
        pltpu.VMEM((K,), jnp.float32),
        pltpu.VMEM((ROWS_PW,), jnp.float32),
        pltpu.VMEM_SHARED((NPAD,), jnp.float32),
    ],
)

_sc_agg = pl.kernel(
    _sc_agg_body,
    out_type=jax.ShapeDtypeStruct((NC, NPAD, HDIM), jnp.float32),
    mesh=_mesh,
    scratch_types=[
        pltpu.VMEM((CPQ, K), jnp.int32),
        pltpu.VMEM((CPQ, K), jnp.int32),
        pltpu.VMEM((K, HDIM), jnp.float32),
        pltpu.VMEM((K, HDIM), jnp.float32),
        pltpu.VMEM_SHARED((N_NODES, HDIM), jnp.float32),
        pltpu.VMEM_SHARED((NPAD, HDIM), jnp.float32),
        pltpu.SemaphoreType.DMA,
        pltpu.SemaphoreType.DMA,
    ],
    compiler_params=pltpu.CompilerParams(use_tc_tiling_on_sc=False),
)

_ROWBLK = 1000
_GRID = N_NODES // _ROWBLK

_tc_dense = pl.pallas_call(
    _tc_dense_body,
    grid=(_GRID,),
    in_specs=[
        pl.BlockSpec((_ROWBLK, DIM), lambda i: (i, 0)),
        pl.BlockSpec((DIM, DIM), lambda i: (0, 0)),
        pl.BlockSpec((1, DIM), lambda i: (0, 0)),
        pl.BlockSpec((1, DIM), lambda i: (0, 0)),
        pl.BlockSpec((1, DIM), lambda i: (0, 0)),
        pl.BlockSpec((1, DIM), lambda i: (0, 0)),
        pl.BlockSpec((1, DIM), lambda i: (0, 0)),
        pl.BlockSpec((DIM, DIM), lambda i: (0, 0)),
        pl.BlockSpec((NC, _ROWBLK, 1), lambda i: (0, i, 0)),
    ],
    out_specs=pl.BlockSpec((NC, _ROWBLK, HDIM), lambda i: (0, i, 0)),
    out_shape=jax.ShapeDtypeStruct((NC, N_NODES, HDIM), jnp.float32),
)

_tc_final = pl.pallas_call(
    _tc_final_body,
    grid=(_GRID,),
    in_specs=[
        pl.BlockSpec((NC, _ROWBLK, HDIM), lambda i: (0, i, 0)),
        pl.BlockSpec((NC, _ROWBLK, HDIM), lambda i: (0, i, 0)),
        pl.BlockSpec((NC, _ROWBLK, 1), lambda i: (0, i, 0)),
        pl.BlockSpec((1, DIM), lambda i: (0, 0)),
    ],
    out_specs=pl.BlockSpec((_ROWBLK, DIM), lambda i: (i, 0)),
    out_shape=jax.ShapeDtypeStruct((N_NODES, DIM), jnp.float32),
)


@jax.jit
def _impl(fea, edges, W1, b1, bn_gamma, bn_beta, bn_mean, bn_var, Wg, bg):
    spad = jnp.zeros((EPAD2 - N_EDGES,), dtype=edges.dtype)
    dpad = jnp.full((EPAD2 - N_EDGES,), N_NODES, dtype=edges.dtype)
    src = jnp.concatenate([edges[0], spad])
    dst = jnp.concatenate([edges[1], dpad])
    dpad_h = jnp.full((EPAD - N_EDGES,), N_NODES, dtype=edges.dtype)
    dst_h = jnp.concatenate([edges[1], dpad_h])

    hist = _sc_hist(dst_h.reshape(NW, NCHUNK, K))
    hist3 = hist[:, :N_NODES].reshape(NC, N_NODES, 1)
    y2 = _tc_dense(fea, W1.T, b1.reshape(1, DIM), bn_gamma.reshape(1, DIM),
                   bn_beta.reshape(1, DIM), bn_mean.reshape(1, DIM),
                   bn_var.reshape(1, DIM), Wg.T, hist3)
    parts = _sc_agg(src.reshape(NS, NPHASE, CPQ, K),
                    dst.reshape(NS, NPHASE, CPQ, K),
                    y2)
    return _tc_final(parts, y2, hist3, bg.reshape(1, DIM))


def kernel(fea, edges, W1, b1, bn_gamma, bn_beta, bn_mean, bn_var, Wg, bg):
    return _impl(fea, edges, W1, b1, bn_gamma, bn_beta, bn_mean, bn_var, Wg, bg)
